# baseline (speedup 1.0000x reference)
import jax
import jax.numpy as jnp
from jax import lax
from jax.experimental import pallas as pl
from jax.experimental.pallas import tpu as pltpu
from jax.experimental.pallas import tpu_sc as plsc

_B = 4096
_S = 50
_D = 100
_DP = 128
_NC = 2
_NS = 16
_NW = _NC * _NS
_EPW = _B // _NW
_CHUNKS = _EPW // 2
_NBUF = 4

_COL_OFFS = (0, 16, 32, 48, 64, 80, 84)


def _sc_pool_body(table_hbm, idx_hbm, out_hbm, idx_v, bufs, stage, sems):
    wid = lax.axis_index("s") * _NC + lax.axis_index("c")

    pltpu.sync_copy(idx_hbm.at[pl.ds(wid * _CHUNKS, _CHUNKS)], idx_v)

    def fire(c, b):
        pltpu.async_copy(table_hbm.at[idx_v.at[c]], bufs[b], sems[b])

    for b in range(_NBUF):
        fire(b, b)

    def accum_chunk(rows_ref, c):
        zeros = jnp.zeros((16,), jnp.float32)

        def body(r, accs):
            new = []
            for e in range(2):
                for k, off in enumerate(_COL_OFFS):
                    v = rows_ref[e * _S + r, pl.ds(off, 16)]
                    new.append(accs[e * 7 + k] + v)
            return tuple(new)

        accs = lax.fori_loop(0, _S, body, (zeros,) * 14)
        for e in range(2):
            for k, off in enumerate(_COL_OFFS):
                stage[2 * c + e, pl.ds(off, 16)] = accs[e * 7 + k]

    def outer(co, _):
        for b in range(_NBUF):
            c = co * _NBUF + b
            pltpu.make_async_copy(table_hbm.at[idx_v.at[0]], bufs[b],
                                  sems[b]).wait()
            accum_chunk(bufs[b], c)
            cnext = c + _NBUF

            @pl.when(cnext < _CHUNKS)
            def _():
                fire(cnext, b)

        return 0

    lax.fori_loop(0, _CHUNKS // _NBUF, outer, 0)

    pltpu.sync_copy(stage, out_hbm.at[pl.ds(wid * _EPW, _EPW)])


def _sc_pool(table_padded, idx2):
    mesh = plsc.VectorSubcoreMesh(core_axis_name="c", subcore_axis_name="s",
                                  num_cores=_NC, num_subcores=_NS)
    kern = pl.kernel(
        _sc_pool_body,
        out_type=jax.ShapeDtypeStruct((_B, _D), jnp.float32),
        mesh=mesh,
        scratch_types=dict(
            idx_v=pltpu.VMEM((_CHUNKS, 2 * _S), jnp.int32),
            bufs=[pltpu.VMEM((2 * _S, _DP), jnp.float32)] * _NBUF,
            stage=pltpu.VMEM((_EPW, _D), jnp.float32),
            sems=[pltpu.SemaphoreType.DMA] * _NBUF,
        ),
    )
    return kern(table_padded, idx2)


def _mlp_body(p_ref, w1_ref, b1_ref, w2_ref, b2_ref, o_ref):
    h = jnp.dot(p_ref[...], w1_ref[...], preferred_element_type=jnp.float32)
    h = jnp.maximum(h + b1_ref[...], 0.0)
    o = jnp.dot(h, w2_ref[...], preferred_element_type=jnp.float32)
    o_ref[...] = o + b2_ref[...]


def _mlp(pooled, w1, b1, w2, b2):
    return pl.pallas_call(
        _mlp_body,
        out_shape=jax.ShapeDtypeStruct((_B, w2.shape[1]), jnp.float32),
    )(pooled, w1, b1, w2, b2)


def kernel(x, table, W1, b1, W2, b2):
    idx2 = x.astype(jnp.int32).reshape(_B // 2, 2 * _S)
    table_padded = jnp.pad(table, ((0, 0), (0, _DP - _D)))
    pooled_sums = _sc_pool(table_padded, idx2)
    w1_scaled = W1 * (1.0 / _S)
    return _mlp(pooled_sums, w1_scaled, b1.reshape(1, -1), W2,
                b2.reshape(1, -1))


# speedup vs baseline: 2.5165x; 2.5165x over previous
import jax
import jax.numpy as jnp
from jax import lax
from jax.experimental import pallas as pl
from jax.experimental.pallas import tpu as pltpu
from jax.experimental.pallas import tpu_sc as plsc

_B = 4096
_S = 50
_D = 100
_DP = 128
_NC = 2
_NS = 16
_NW = _NC * _NS
_EPW = _B // _NW
_CHUNKS = _EPW // 2
_NBUF = 4

_COL_OFFS = (0, 16, 32, 48, 64, 80, 84)


def _sc_pool_body(table_hbm, idx_hbm, out_hbm, idx_v, bufs, stage, sems):
    wid = lax.axis_index("s") * _NC + lax.axis_index("c")

    pltpu.sync_copy(idx_hbm.at[pl.ds(wid * _CHUNKS, _CHUNKS)], idx_v)

    def fire(c, b):
        pltpu.async_copy(table_hbm.at[idx_v.at[c]], bufs[b], sems[b])

    for b in range(_NBUF):
        fire(b, b)

    def accum_chunk(rows_ref, c):
        zeros = jnp.zeros((16,), jnp.float32)

        def body(r, accs):
            new = []
            for e in range(2):
                for k, off in enumerate(_COL_OFFS):
                    v = rows_ref[e * _S + r, pl.ds(off, 16)]
                    new.append(accs[e * 7 + k] + v)
            return tuple(new)

        accs = lax.fori_loop(0, _S, body, (zeros,) * 14)
        for e in range(2):
            for k, off in enumerate(_COL_OFFS):
                stage[2 * c + e, pl.ds(off, 16)] = accs[e * 7 + k]

    def outer(co, _):
        for b in range(_NBUF):
            c = co * _NBUF + b
            pltpu.make_async_copy(table_hbm.at[idx_v.at[0]], bufs[b],
                                  sems[b]).wait()
            accum_chunk(bufs[b], c)
            cnext = c + _NBUF

            @pl.when(cnext < _CHUNKS)
            def _():
                fire(cnext, b)

        return 0

    lax.fori_loop(0, _CHUNKS // _NBUF, outer, 0)

    pltpu.sync_copy(stage, out_hbm.at[pl.ds(wid * _EPW, _EPW)])


def _sc_pool(table_padded, idx2):
    mesh = plsc.VectorSubcoreMesh(core_axis_name="c", subcore_axis_name="s",
                                  num_cores=_NC, num_subcores=_NS)
    kern = pl.kernel(
        _sc_pool_body,
        out_type=jax.ShapeDtypeStruct((_B, _D), jnp.float32),
        mesh=mesh,
        scratch_types=dict(
            idx_v=pltpu.VMEM((_CHUNKS, 2 * _S), jnp.int32),
            bufs=[pltpu.VMEM((2 * _S, _DP), jnp.float32)] * _NBUF,
            stage=pltpu.VMEM((_EPW, _D), jnp.float32),
            sems=[pltpu.SemaphoreType.DMA] * _NBUF,
        ),
    )
    return kern(table_padded, idx2)


def _pad_body(t_ref, o_ref):
    blk = t_ref[...]
    o_ref[...] = jnp.concatenate(
        [blk, jnp.zeros((blk.shape[0], _DP - _D), jnp.float32)], axis=1)


def _pad_table(table):
    rows_per_blk = 8000
    grid = table.shape[0] // rows_per_blk
    return pl.pallas_call(
        _pad_body,
        grid=(grid,),
        in_specs=[pl.BlockSpec((rows_per_blk, _D), lambda i: (i, 0))],
        out_specs=pl.BlockSpec((rows_per_blk, _DP), lambda i: (i, 0)),
        out_shape=jax.ShapeDtypeStruct((table.shape[0], _DP), jnp.float32),
    )(table)


def _mlp_body(p_ref, w1_ref, b1_ref, w2_ref, b2_ref, o_ref):
    h = jnp.dot(p_ref[...], w1_ref[...], preferred_element_type=jnp.float32)
    h = jnp.maximum(h + b1_ref[...], 0.0)
    o = jnp.dot(h, w2_ref[...], preferred_element_type=jnp.float32)
    o_ref[...] = o + b2_ref[...]


def _mlp(pooled, w1, b1, w2, b2):
    return pl.pallas_call(
        _mlp_body,
        out_shape=jax.ShapeDtypeStruct((_B, w2.shape[1]), jnp.float32),
    )(pooled, w1, b1, w2, b2)


def kernel(x, table, W1, b1, W2, b2):
    idx2 = x.astype(jnp.int32).reshape(_B // 2, 2 * _S)
    table_padded = _pad_table(table)
    pooled_sums = _sc_pool(table_padded, idx2)
    w1_scaled = W1 * (1.0 / _S)
    return _mlp(pooled_sums, w1_scaled, b1.reshape(1, -1), W2,
                b2.reshape(1, -1))


# speedup vs baseline: 2.5192x; 1.0011x over previous
import jax
import jax.numpy as jnp
from jax import lax
from jax.experimental import pallas as pl
from jax.experimental.pallas import tpu as pltpu
from jax.experimental.pallas import tpu_sc as plsc

_B = 4096
_S = 50
_D = 100
_DP = 128
_NC = 2
_NS = 16
_NW = _NC * _NS
_EPW = _B // _NW
_CHUNKS = _EPW // 2
_NBUF = 4

_COL_OFFS = (0, 16, 32, 48, 64, 80, 84)


def _sc_pool_body(table_hbm, idx_hbm, out_hbm, idx_v, bufs, stage, sems):
    wid = lax.axis_index("s") * _NC + lax.axis_index("c")

    pltpu.sync_copy(idx_hbm.at[pl.ds(wid * _CHUNKS, _CHUNKS)], idx_v)

    def fire(c, b):
        pltpu.async_copy(table_hbm.at[idx_v.at[c]], bufs[b], sems[b])

    for b in range(_NBUF):
        fire(b, b)

    def accum_chunk(rows_ref, c):
        zeros = jnp.zeros((16,), jnp.float32)

        def body(r, accs):
            new = []
            for e in range(2):
                for k, off in enumerate(_COL_OFFS):
                    v = rows_ref[e * _S + r, pl.ds(off, 16)]
                    new.append(accs[e * 7 + k] + v)
            return tuple(new)

        accs = lax.fori_loop(0, _S, body, (zeros,) * 14)
        for e in range(2):
            for k, off in enumerate(_COL_OFFS):
                stage[2 * c + e, pl.ds(off, 16)] = accs[e * 7 + k]

    def outer(co, _):
        for b in range(_NBUF):
            c = co * _NBUF + b
            pltpu.make_async_copy(table_hbm.at[idx_v.at[0]], bufs[b],
                                  sems[b]).wait()
            accum_chunk(bufs[b], c)
            cnext = c + _NBUF

            @pl.when(cnext < _CHUNKS)
            def _():
                fire(cnext, b)

        return 0

    lax.fori_loop(0, _CHUNKS // _NBUF, outer, 0)

    pltpu.sync_copy(stage, out_hbm.at[pl.ds(wid * _EPW, _EPW)])


def _sc_pool(table_padded, idx2):
    mesh = plsc.VectorSubcoreMesh(core_axis_name="c", subcore_axis_name="s",
                                  num_cores=_NC, num_subcores=_NS)
    kern = pl.kernel(
        _sc_pool_body,
        out_type=jax.ShapeDtypeStruct((_B, _D), jnp.float32),
        mesh=mesh,
        scratch_types=dict(
            idx_v=pltpu.VMEM((_CHUNKS, 2 * _S), jnp.int32),
            bufs=[pltpu.VMEM((2 * _S, _DP), jnp.float32)] * _NBUF,
            stage=pltpu.VMEM((_EPW, _D), jnp.float32),
            sems=[pltpu.SemaphoreType.DMA] * _NBUF,
        ),
    )
    return kern(table_padded, idx2)


def _pad_body(t_ref, o_ref):
    blk = t_ref[...]
    o_ref[...] = jnp.concatenate(
        [blk, jnp.zeros((blk.shape[0], _DP - _D), jnp.float32)], axis=1)


def _pad_table(table):
    rows_per_blk = 20000
    grid = table.shape[0] // rows_per_blk
    return pl.pallas_call(
        _pad_body,
        grid=(grid,),
        in_specs=[pl.BlockSpec((rows_per_blk, _D), lambda i: (i, 0))],
        out_specs=pl.BlockSpec((rows_per_blk, _DP), lambda i: (i, 0)),
        out_shape=jax.ShapeDtypeStruct((table.shape[0], _DP), jnp.float32),
    )(table)


def _mlp_body(p_ref, w1_ref, b1_ref, w2_ref, b2_ref, o_ref):
    h = jnp.dot(p_ref[...], w1_ref[...], preferred_element_type=jnp.float32)
    h = jnp.maximum(h + b1_ref[...], 0.0)
    o = jnp.dot(h, w2_ref[...], preferred_element_type=jnp.float32)
    o_ref[...] = o + b2_ref[...]


def _mlp(pooled, w1, b1, w2, b2):
    return pl.pallas_call(
        _mlp_body,
        out_shape=jax.ShapeDtypeStruct((_B, w2.shape[1]), jnp.float32),
    )(pooled, w1, b1, w2, b2)


def kernel(x, table, W1, b1, W2, b2):
    idx2 = x.astype(jnp.int32).reshape(_B // 2, 2 * _S)
    table_padded = _pad_table(table)
    pooled_sums = _sc_pool(table_padded, idx2)
    w1_scaled = W1 * (1.0 / _S)
    return _mlp(pooled_sums, w1_scaled, b1.reshape(1, -1), W2,
                b2.reshape(1, -1))


# speedup vs baseline: 2.5222x; 1.0012x over previous
import jax
import jax.numpy as jnp
from jax import lax
from jax.experimental import pallas as pl
from jax.experimental.pallas import tpu as pltpu
from jax.experimental.pallas import tpu_sc as plsc

_B = 4096
_S = 50
_D = 100
_DP = 128
_NC = 2
_NS = 16
_NW = _NC * _NS
_EPW = _B // _NW
_CHUNKS = _EPW // 2
_NBUF = 4

_COL_OFFS = (0, 16, 32, 48, 64, 80, 84)


def _sc_pool_body(table_hbm, idx_hbm, out_hbm, idx_v, bufs, stage, sems):
    wid = lax.axis_index("s") * _NC + lax.axis_index("c")

    pltpu.sync_copy(idx_hbm.at[pl.ds(wid * _CHUNKS, _CHUNKS)], idx_v)

    def fire(c, b):
        pltpu.async_copy(table_hbm.at[idx_v.at[c]], bufs[b], sems[b])

    for b in range(_NBUF):
        fire(b, b)

    def accum_chunk(rows_ref, c):
        zeros = jnp.zeros((16,), jnp.float32)

        def body(r, accs):
            new = []
            for e in range(2):
                for k, off in enumerate(_COL_OFFS):
                    v = rows_ref[e * _S + r, pl.ds(off, 16)]
                    new.append(accs[e * 7 + k] + v)
            return tuple(new)

        accs = lax.fori_loop(0, _S, body, (zeros,) * 14)
        for e in range(2):
            for k, off in enumerate(_COL_OFFS):
                stage[2 * c + e, pl.ds(off, 16)] = accs[e * 7 + k]

    def outer(co, _):
        for b in range(_NBUF):
            c = co * _NBUF + b
            pltpu.make_async_copy(table_hbm.at[idx_v.at[0]], bufs[b],
                                  sems[b]).wait()
            accum_chunk(bufs[b], c)
            cnext = c + _NBUF

            @pl.when(cnext < _CHUNKS)
            def _():
                fire(cnext, b)

        return 0

    lax.fori_loop(0, _CHUNKS // _NBUF, outer, 0)

    pltpu.sync_copy(stage, out_hbm.at[pl.ds(wid * _EPW, _EPW)])


def _sc_pool(table_padded, idx2):
    mesh = plsc.VectorSubcoreMesh(core_axis_name="c", subcore_axis_name="s",
                                  num_cores=_NC, num_subcores=_NS)
    kern = pl.kernel(
        _sc_pool_body,
        out_type=jax.ShapeDtypeStruct((_B, _D), jnp.float32),
        mesh=mesh,
        scratch_types=dict(
            idx_v=pltpu.VMEM((_CHUNKS, 2 * _S), jnp.int32),
            bufs=[pltpu.VMEM((2 * _S, _DP), jnp.float32)] * _NBUF,
            stage=pltpu.VMEM((_EPW, _D), jnp.float32),
            sems=[pltpu.SemaphoreType.DMA] * _NBUF,
        ),
    )
    return kern(table_padded, idx2)


_PAD_R = 10000
_PAD_RING = 4


def _pad_body(in_hbm, out_hbm, s100, s128, in_sems, out_sems):
    nb = in_hbm.shape[0] // _PAD_R

    def in_dma(k, j):
        return pltpu.make_async_copy(
            in_hbm.at[pl.ds(k * _PAD_R, _PAD_R)], s100[j], in_sems[j])

    def out_dma(k, j):
        return pltpu.make_async_copy(
            s128[j], out_hbm.at[pl.ds(k * _PAD_R, _PAD_R)], out_sems[j])

    for j in range(_PAD_RING):
        in_dma(j, j).start()

    zeros = jnp.zeros((_PAD_R, _DP - _D), jnp.float32)

    def outer(ko, _):
        for j in range(_PAD_RING):
            k = ko * _PAD_RING + j

            @pl.when(k >= _PAD_RING)
            def _():
                out_dma(k - _PAD_RING, j).wait()

            in_dma(k, j).wait()
            s128[j][...] = jnp.concatenate([s100[j][...], zeros], axis=1)

            knext = k + _PAD_RING

            @pl.when(knext < nb)
            def _():
                in_dma(knext, j).start()

            out_dma(k, j).start()
        return 0

    lax.fori_loop(0, nb // _PAD_RING, outer, 0)
    for j in range(_PAD_RING):
        out_dma(nb - _PAD_RING + j, j).wait()


def _pad_table(table):
    return pl.pallas_call(
        _pad_body,
        in_specs=[pl.BlockSpec(memory_space=pl.ANY)],
        out_specs=pl.BlockSpec(memory_space=pl.ANY),
        out_shape=jax.ShapeDtypeStruct((table.shape[0], _DP), jnp.float32),
        scratch_shapes=[
            [pltpu.VMEM((_PAD_R, _D), jnp.float32)] * _PAD_RING,
            [pltpu.VMEM((_PAD_R, _DP), jnp.float32)] * _PAD_RING,
            [pltpu.SemaphoreType.DMA] * _PAD_RING,
            [pltpu.SemaphoreType.DMA] * _PAD_RING,
        ],
    )(table)


def _mlp_body(p_ref, w1_ref, b1_ref, w2_ref, b2_ref, o_ref):
    h = jnp.dot(p_ref[...], w1_ref[...], preferred_element_type=jnp.float32)
    h = jnp.maximum(h + b1_ref[...], 0.0)
    o = jnp.dot(h, w2_ref[...], preferred_element_type=jnp.float32)
    o_ref[...] = o + b2_ref[...]


def _mlp(pooled, w1, b1, w2, b2):
    return pl.pallas_call(
        _mlp_body,
        out_shape=jax.ShapeDtypeStruct((_B, w2.shape[1]), jnp.float32),
    )(pooled, w1, b1, w2, b2)


def kernel(x, table, W1, b1, W2, b2):
    idx2 = x.astype(jnp.int32).reshape(_B // 2, 2 * _S)
    table_padded = _pad_table(table)
    pooled_sums = _sc_pool(table_padded, idx2)
    w1_scaled = W1 * (1.0 / _S)
    return _mlp(pooled_sums, w1_scaled, b1.reshape(1, -1), W2,
                b2.reshape(1, -1))
